# Initial kernel scaffold; baseline (speedup 1.0000x reference)
import jax
import jax.numpy as jnp
from jax.experimental import pallas as pl


def kernel(x, node_embeddings, W_gate, b_gate, W_update, b_update):
    raise NotImplementedError("write your pallas kernel here")


# speedup vs baseline: 1.8222x; 1.8222x over previous
import jax
import jax.numpy as jnp
from jax.experimental import pallas as pl
from jax.experimental.pallas import tpu as pltpu

N = 1024
H = 32
B = 8
T = 8
EMB = 10
EPAD = 16


def _gcn_gru_body(x_ref, ep_ref, ept_ref, wsv_g_ref, wx_g_ref, wsv_u_ref,
                  wx_u_ref, bg_ref, bu_ref, out_ref,
                  s_sc, state_sc, biasg_sc, biasu_sc):
    t = pl.program_id(0)

    @pl.when(t == 0)
    def _init():
        e = ep_ref[...]
        g = jnp.dot(e, ept_ref[...], preferred_element_type=jnp.float32)
        g = jnp.maximum(g, 0.0)
        m = jnp.max(g, axis=1, keepdims=True)
        ex = jnp.exp(g - m)
        s_sc[...] = ex / jnp.sum(ex, axis=1, keepdims=True)
        biasg_sc[...] = jnp.dot(e, bg_ref[...], preferred_element_type=jnp.float32)
        biasu_sc[...] = jnp.dot(e, bu_ref[...], preferred_element_type=jnp.float32)
        state_sc[...] = jnp.zeros_like(state_sc)

    s = s_sc[...]
    e = ep_ref[...]
    xt = x_ref[0]
    st = state_sc[...]

    u = jnp.dot(s, xt, preferred_element_type=jnp.float32)
    v = jnp.dot(s, st, preferred_element_type=jnp.float32)

    wsv_g = wsv_g_ref[...]
    wx_g = wx_g_ref[...]
    biasg = biasg_sc[...]

    cz_list = []
    r_list = []
    st_list = []
    for b in range(B):
        xt_b = xt[:, b:b + 1]
        u_b = u[:, b:b + 1]
        st_b = st[:, b * H:(b + 1) * H]
        v_b = v[:, b * H:(b + 1) * H]
        sv = jnp.concatenate([st_b, v_b], axis=1)
        y = jnp.dot(sv, wsv_g, preferred_element_type=jnp.float32)
        y = y + xt_b * wx_g[0:1, :] + u_b * wx_g[1:2, :]
        acc = biasg
        for d in range(EMB):
            acc = acc + e[:, d:d + 1] * y[:, d * 2 * H:(d + 1) * 2 * H]
        zr = jax.nn.sigmoid(acc)
        z = zr[:, :H]
        r = zr[:, H:]
        cz_list.append(z * st_b)
        r_list.append(r)
        st_list.append(st_b)

    cz = jnp.concatenate(cz_list, axis=1)
    v2 = jnp.dot(s, cz, preferred_element_type=jnp.float32)

    wsv_u = wsv_u_ref[...]
    wx_u = wx_u_ref[...]
    biasu = biasu_sc[...]

    ns_list = []
    for b in range(B):
        xt_b = xt[:, b:b + 1]
        u_b = u[:, b:b + 1]
        cz_b = cz_list[b]
        v2_b = v2[:, b * H:(b + 1) * H]
        sv = jnp.concatenate([cz_b, v2_b], axis=1)
        y = jnp.dot(sv, wsv_u, preferred_element_type=jnp.float32)
        y = y + xt_b * wx_u[0:1, :] + u_b * wx_u[1:2, :]
        acc = biasu
        for d in range(EMB):
            acc = acc + e[:, d:d + 1] * y[:, d * H:(d + 1) * H]
        hc = jnp.tanh(acc)
        r = r_list[b]
        ns_list.append(r * st_list[b] + (1.0 - r) * hc)

    ns = jnp.concatenate(ns_list, axis=1)
    state_sc[...] = ns
    out_ref[0] = ns


def kernel(x, node_embeddings, W_gate, b_gate, W_update, b_update):
    xt = jnp.transpose(x[..., 0], (1, 2, 0))
    e_pad = jnp.pad(node_embeddings, ((0, 0), (0, EPAD - EMB)))
    e_pad_t = e_pad.T

    def _prep(wpool):
        o = wpool.shape[-1]
        w0s = jnp.transpose(wpool[:, 0, 1:, :], (1, 0, 2)).reshape(H, EMB * o)
        w1s = jnp.transpose(wpool[:, 1, 1:, :], (1, 0, 2)).reshape(H, EMB * o)
        wsv = jnp.concatenate([w0s, w1s], axis=0)
        wx = jnp.stack([wpool[:, 0, 0, :].reshape(EMB * o),
                        wpool[:, 1, 0, :].reshape(EMB * o)], axis=0)
        return wsv, wx

    wsv_g, wx_g = _prep(W_gate)
    wsv_u, wx_u = _prep(W_update)
    bg_pad = jnp.pad(b_gate, ((0, EPAD - EMB), (0, 0)))
    bu_pad = jnp.pad(b_update, ((0, EPAD - EMB), (0, 0)))

    const = lambda *_: (0, 0)
    out = pl.pallas_call(
        _gcn_gru_body,
        grid=(T,),
        in_specs=[
            pl.BlockSpec((1, N, B), lambda t: (t, 0, 0)),
            pl.BlockSpec((N, EPAD), const),
            pl.BlockSpec((EPAD, N), const),
            pl.BlockSpec((2 * H, EMB * 2 * H), const),
            pl.BlockSpec((2, EMB * 2 * H), const),
            pl.BlockSpec((2 * H, EMB * H), const),
            pl.BlockSpec((2, EMB * H), const),
            pl.BlockSpec((EPAD, 2 * H), const),
            pl.BlockSpec((EPAD, H), const),
        ],
        out_specs=pl.BlockSpec((1, N, B * H), lambda t: (t, 0, 0)),
        out_shape=jax.ShapeDtypeStruct((T, N, B * H), jnp.float32),
        scratch_shapes=[
            pltpu.VMEM((N, N), jnp.float32),
            pltpu.VMEM((N, B * H), jnp.float32),
            pltpu.VMEM((N, 2 * H), jnp.float32),
            pltpu.VMEM((N, H), jnp.float32),
        ],
    )(xt, e_pad, e_pad_t, wsv_g, wx_g, wsv_u, wx_u, bg_pad, bu_pad)

    layer_output = jnp.transpose(out.reshape(T, N, B, H), (2, 0, 1, 3))
    return (layer_output, layer_output[:, -1])


# speedup vs baseline: 2.8083x; 1.5412x over previous
import jax
import jax.numpy as jnp
from jax.experimental import pallas as pl
from jax.experimental.pallas import tpu as pltpu

N = 1024
H = 32
B = 8
T = 8
EMB = 10
EPAD = 16
OG = 2 * H
OU = H


def _gcn_gru_body(x_ref, x2_ref, ep_ref, ept_ref, mg_ref, mu_ref,
                  wsvx_g_ref, wsvx_u_ref, bg_ref, bu_ref, out_ref,
                  s_sc, state_sc, u_sc, cz_sc, r_sc,
                  biasg_sc, biasu_sc, ebcg_sc, ebcu_sc):
    t = pl.program_id(0)

    @pl.when(t == 0)
    def _init():
        e = ep_ref[...]
        g = jnp.dot(e, ept_ref[...], preferred_element_type=jnp.float32)
        g = jnp.maximum(g, 0.0)
        m = jnp.max(g, axis=1, keepdims=True)
        ex = jnp.exp(g - m)
        s = ex / jnp.sum(ex, axis=1, keepdims=True)
        s_sc[...] = s
        biasg_sc[...] = jnp.dot(e, bg_ref[...], preferred_element_type=jnp.float32)
        biasu_sc[...] = jnp.dot(e, bu_ref[...], preferred_element_type=jnp.float32)
        ebcg_sc[...] = jnp.dot(e, mg_ref[...], preferred_element_type=jnp.float32)
        ebcu_sc[...] = jnp.dot(e, mu_ref[...], preferred_element_type=jnp.float32)
        uall = jnp.dot(s, x2_ref[...], preferred_element_type=jnp.float32)
        for tt in range(T):
            u_sc[tt] = uall[:, tt * B:(tt + 1) * B]
        state_sc[...] = jnp.zeros_like(state_sc)

    s = s_sc[...]
    xt = x_ref[0]
    u = u_sc[t]
    st = state_sc[...]

    v = jnp.dot(s, st, preferred_element_type=jnp.float32)

    wsvx_g = wsvx_g_ref[...]
    biasg = biasg_sc[...]
    ebcg = ebcg_sc[...]

    for b in range(B):
        xt_b = xt[:, b:b + 1]
        u_b = u[:, b:b + 1]
        st_b = st[:, b * H:(b + 1) * H]
        v_b = v[:, b * H:(b + 1) * H]
        svx = jnp.concatenate([st_b, v_b, xt_b, u_b], axis=1)
        y = jnp.dot(svx, wsvx_g, preferred_element_type=jnp.float32)
        acc = y[:, 0:128] * ebcg[:, 0:128]
        for j in range(1, 5):
            acc = acc + y[:, j * 128:(j + 1) * 128] * ebcg[:, j * 128:(j + 1) * 128]
        zr = jax.nn.sigmoid(acc[:, :OG] + acc[:, OG:] + biasg)
        z = zr[:, :H]
        r = zr[:, H:]
        cz_sc[:, b * H:(b + 1) * H] = z * st_b
        r_sc[:, b * H:(b + 1) * H] = r

    cz = cz_sc[...]
    v2 = jnp.dot(s, cz, preferred_element_type=jnp.float32)

    wsvx_u = wsvx_u_ref[...]
    biasu = biasu_sc[...]
    ebcu = ebcu_sc[...]
    rr = r_sc[...]

    for b in range(B):
        xt_b = xt[:, b:b + 1]
        u_b = u[:, b:b + 1]
        st_b = st[:, b * H:(b + 1) * H]
        cz_b = cz[:, b * H:(b + 1) * H]
        v2_b = v2[:, b * H:(b + 1) * H]
        svx = jnp.concatenate([cz_b, v2_b, xt_b, u_b], axis=1)
        y = jnp.dot(svx, wsvx_u, preferred_element_type=jnp.float32)
        acc = y[:, 0:128] * ebcu[:, 0:128] + y[:, 128:256] * ebcu[:, 128:256]
        accb = y[:, 256:320] * ebcu[:, 256:320]
        acc64 = acc[:, :64] + acc[:, 64:] + accb
        hc = jnp.tanh(acc64[:, :OU] + acc64[:, OU:] + biasu)
        r = rr[:, b * H:(b + 1) * H]
        ns = r * st_b + (1.0 - r) * hc
        state_sc[:, b * H:(b + 1) * H] = ns
        out_ref[0, :, b * H:(b + 1) * H] = ns


def kernel(x, node_embeddings, W_gate, b_gate, W_update, b_update):
    x0 = x[..., 0]
    xt = jnp.transpose(x0, (1, 2, 0))
    x2 = jnp.transpose(x0, (2, 1, 0)).reshape(N, T * B)
    e_pad = jnp.pad(node_embeddings, ((0, 0), (0, EPAD - EMB)))
    e_pad_t = e_pad.T

    def _prep(wpool):
        o = wpool.shape[-1]
        w0s = jnp.transpose(wpool[:, 0, 1:, :], (1, 0, 2)).reshape(H, EMB * o)
        w1s = jnp.transpose(wpool[:, 1, 1:, :], (1, 0, 2)).reshape(H, EMB * o)
        w0x = wpool[:, 0, 0, :].reshape(1, EMB * o)
        w1x = wpool[:, 1, 0, :].reshape(1, EMB * o)
        return jnp.concatenate([w0s, w1s, w0x, w1x], axis=0)

    wsvx_g = _prep(W_gate)
    wsvx_u = _prep(W_update)
    bg_pad = jnp.pad(b_gate, ((0, EPAD - EMB), (0, 0)))
    bu_pad = jnp.pad(b_update, ((0, EPAD - EMB), (0, 0)))
    mg = jnp.pad(jnp.kron(jnp.eye(EMB, dtype=jnp.float32),
                          jnp.ones((1, OG), jnp.float32)),
                 ((0, EPAD - EMB), (0, 0)))
    mu = jnp.pad(jnp.kron(jnp.eye(EMB, dtype=jnp.float32),
                          jnp.ones((1, OU), jnp.float32)),
                 ((0, EPAD - EMB), (0, 0)))

    const = lambda *_: (0, 0)
    out = pl.pallas_call(
        _gcn_gru_body,
        grid=(T,),
        in_specs=[
            pl.BlockSpec((1, N, B), lambda t: (t, 0, 0)),
            pl.BlockSpec((N, T * B), const),
            pl.BlockSpec((N, EPAD), const),
            pl.BlockSpec((EPAD, N), const),
            pl.BlockSpec((EPAD, EMB * OG), const),
            pl.BlockSpec((EPAD, EMB * OU), const),
            pl.BlockSpec((2 * H + 2, EMB * OG), const),
            pl.BlockSpec((2 * H + 2, EMB * OU), const),
            pl.BlockSpec((EPAD, OG), const),
            pl.BlockSpec((EPAD, OU), const),
        ],
        out_specs=pl.BlockSpec((1, N, B * H), lambda t: (t, 0, 0)),
        out_shape=jax.ShapeDtypeStruct((T, N, B * H), jnp.float32),
        scratch_shapes=[
            pltpu.VMEM((N, N), jnp.float32),
            pltpu.VMEM((N, B * H), jnp.float32),
            pltpu.VMEM((T, N, B), jnp.float32),
            pltpu.VMEM((N, B * H), jnp.float32),
            pltpu.VMEM((N, B * H), jnp.float32),
            pltpu.VMEM((N, OG), jnp.float32),
            pltpu.VMEM((N, OU), jnp.float32),
            pltpu.VMEM((N, EMB * OG), jnp.float32),
            pltpu.VMEM((N, EMB * OU), jnp.float32),
        ],
    )(xt, x2, e_pad, e_pad_t, mg, mu, wsvx_g, wsvx_u, bg_pad, bu_pad)

    layer_output = jnp.transpose(out.reshape(T, N, B, H), (2, 0, 1, 3))
    return (layer_output, layer_output[:, -1])


# speedup vs baseline: 5.7633x; 2.0522x over previous
import jax
import jax.numpy as jnp
from jax.experimental import pallas as pl
from jax.experimental.pallas import tpu as pltpu

N = 1024
H = 32
B = 8
T = 8
EMB = 10
EPAD = 16
OG = 2 * H
OU = H
KR = 72


def _gcn_gru_body(x2t_ref, ep_ref, ept_ref, mgt_ref, mut_ref,
                  wgt_ref, wut_ref, out_ref,
                  st_sc, state_sc, tails_sc, cz_sc, r_sc,
                  ebcg_sc, ebcu_sc):
    t = pl.program_id(0)

    @pl.when(t == 0)
    def _init():
        e = ep_ref[...]
        g = jnp.dot(e, ept_ref[...], preferred_element_type=jnp.float32)
        g = jnp.maximum(g, 0.0)
        m = jnp.max(g, axis=0, keepdims=True)
        ex = jnp.exp(g - m)
        s_t = ex / jnp.sum(ex, axis=0, keepdims=True)
        st_sc[...] = s_t
        ebcg_sc[...] = jnp.dot(mgt_ref[...], ept_ref[...],
                               preferred_element_type=jnp.float32)
        ebcu_sc[...] = jnp.dot(mut_ref[...], ept_ref[...],
                               preferred_element_type=jnp.float32)
        x2t = x2t_ref[...]
        uall = jnp.dot(x2t, s_t, preferred_element_type=jnp.float32)
        tails_sc[...] = jnp.zeros_like(tails_sc)
        one_row = jnp.ones((1, N), jnp.float32)
        for tt in range(T):
            for b in range(B):
                i = tt * B + b
                tails_sc[tt, b * 8 + 0:b * 8 + 1] = x2t[i:i + 1]
                tails_sc[tt, b * 8 + 1:b * 8 + 2] = uall[i:i + 1]
                tails_sc[tt, b * 8 + 2:b * 8 + 3] = one_row
        state_sc[...] = jnp.zeros_like(state_sc)

    s_t = st_sc[...]
    stt = state_sc[...]
    tails = tails_sc[t]

    v = jnp.dot(stt, s_t, preferred_element_type=jnp.float32)

    wgt = wgt_ref[...]
    ebcg = ebcg_sc[...]

    for b in range(B):
        st_b = stt[b * H:(b + 1) * H]
        v_b = v[b * H:(b + 1) * H]
        tail_b = tails[b * 8:(b + 1) * 8]
        svx = jnp.concatenate([st_b, v_b, tail_b], axis=0)
        y = jnp.dot(wgt, svx, preferred_element_type=jnp.float32)
        acc = y[0:OG] * ebcg[0:OG]
        for d in range(1, EMB):
            acc = acc + y[d * OG:(d + 1) * OG] * ebcg[d * OG:(d + 1) * OG]
        zr = jax.nn.sigmoid(acc)
        z = zr[:H]
        r = zr[H:]
        cz_sc[b * H:(b + 1) * H] = z * st_b
        r_sc[b * H:(b + 1) * H] = r

    cz = cz_sc[...]
    v2 = jnp.dot(cz, s_t, preferred_element_type=jnp.float32)

    wut = wut_ref[...]
    ebcu = ebcu_sc[...]
    rr = r_sc[...]

    for b in range(B):
        st_b = stt[b * H:(b + 1) * H]
        cz_b = cz[b * H:(b + 1) * H]
        v2_b = v2[b * H:(b + 1) * H]
        tail_b = tails[b * 8:(b + 1) * 8]
        svx = jnp.concatenate([cz_b, v2_b, tail_b], axis=0)
        y = jnp.dot(wut, svx, preferred_element_type=jnp.float32)
        acc = y[0:OU] * ebcu[0:OU]
        for d in range(1, EMB):
            acc = acc + y[d * OU:(d + 1) * OU] * ebcu[d * OU:(d + 1) * OU]
        hc = jnp.tanh(acc)
        r = rr[b * H:(b + 1) * H]
        ns = r * st_b + (1.0 - r) * hc
        state_sc[b * H:(b + 1) * H] = ns
        out_ref[0, b * H:(b + 1) * H, :] = ns


def kernel(x, node_embeddings, W_gate, b_gate, W_update, b_update):
    x0 = x[..., 0]
    x2t = jnp.transpose(x0, (1, 0, 2)).reshape(T * B, N)
    e_pad = jnp.pad(node_embeddings, ((0, 0), (0, EPAD - EMB)))
    e_pad_t = e_pad.T

    def _prep(wpool, bpool):
        o = wpool.shape[-1]
        w0s = jnp.transpose(wpool[:, 0, 1:, :], (1, 0, 2)).reshape(H, EMB * o)
        w1s = jnp.transpose(wpool[:, 1, 1:, :], (1, 0, 2)).reshape(H, EMB * o)
        w0x = wpool[:, 0, 0, :].reshape(1, EMB * o)
        w1x = wpool[:, 1, 0, :].reshape(1, EMB * o)
        bfl = bpool.reshape(1, EMB * o)
        wk = jnp.concatenate([w0s, w1s, w0x, w1x, bfl], axis=0)
        wk = jnp.pad(wk, ((0, KR - wk.shape[0]), (0, 0)))
        return wk.T

    wgt = _prep(W_gate, b_gate)
    wut = _prep(W_update, b_update)
    mgt = jnp.pad(jnp.kron(jnp.eye(EMB, dtype=jnp.float32),
                           jnp.ones((OG, 1), jnp.float32)),
                  ((0, 0), (0, EPAD - EMB)))
    mut = jnp.pad(jnp.kron(jnp.eye(EMB, dtype=jnp.float32),
                           jnp.ones((OU, 1), jnp.float32)),
                  ((0, 0), (0, EPAD - EMB)))

    const = lambda *_: (0, 0)
    out = pl.pallas_call(
        _gcn_gru_body,
        grid=(T,),
        in_specs=[
            pl.BlockSpec((T * B, N), const),
            pl.BlockSpec((N, EPAD), const),
            pl.BlockSpec((EPAD, N), const),
            pl.BlockSpec((EMB * OG, EPAD), const),
            pl.BlockSpec((EMB * OU, EPAD), const),
            pl.BlockSpec((EMB * OG, KR), const),
            pl.BlockSpec((EMB * OU, KR), const),
        ],
        out_specs=pl.BlockSpec((1, B * H, N), lambda t: (t, 0, 0)),
        out_shape=jax.ShapeDtypeStruct((T, B * H, N), jnp.float32),
        scratch_shapes=[
            pltpu.VMEM((N, N), jnp.float32),
            pltpu.VMEM((B * H, N), jnp.float32),
            pltpu.VMEM((T, B * 8, N), jnp.float32),
            pltpu.VMEM((B * H, N), jnp.float32),
            pltpu.VMEM((B * H, N), jnp.float32),
            pltpu.VMEM((EMB * OG, N), jnp.float32),
            pltpu.VMEM((EMB * OU, N), jnp.float32),
        ],
    )(x2t, e_pad, e_pad_t, mgt, mut, wgt, wut)

    layer_output = jnp.transpose(out.reshape(T, B, H, N), (1, 0, 3, 2))
    return (layer_output, layer_output[:, -1])
